# baseline (speedup 1.0000x reference)
import functools

import jax
import jax.numpy as jnp
from jax import lax
from jax.experimental import pallas as pl
from jax.experimental.pallas import tpu as pltpu
from jax.experimental.pallas import tpu_sc as plsc

C = 15
EMBED = 1024
B = 1024
HALF = B // 2
BP = 8
ROWS = BP * C
GRID = HALF // BP
F1 = 4096


def _lrelu(x):
    return jnp.where(x >= 0, x, 0.2 * x)


def _elu(x):
    return jnp.where(x > 0, x, jnp.exp(x) - 1.0)


_EPAD = 448


def _sc_hist_body(ei_hbm, out_hbm, idx_v, cnt_v, sem):
    wid = lax.axis_index("s") * 2 + lax.axis_index("c")

    @pl.when(wid == 0)
    def _():
        pltpu.async_copy(ei_hbm, idx_v, sem).wait()
        zeros = jnp.zeros((16,), jnp.float32)
        ones = jnp.full((16,), 1.0, jnp.float32)
        accs = [zeros] * C
        for j in range(_EPAD // 16):
            chunk = idx_v[pl.ds(j * 16, 16)]
            for v in range(C):
                accs[v] = accs[v] + jnp.where(chunk == v, ones, zeros)
        for v in range(C):
            cnt_v[v] = accs[v]
        pltpu.async_copy(cnt_v, out_hbm, sem).wait()


def _sc_histogram(ei_flat_padded):
    mesh = plsc.VectorSubcoreMesh(core_axis_name="c", subcore_axis_name="s")
    kern = functools.partial(
        pl.kernel,
        mesh=mesh,
        out_type=jax.ShapeDtypeStruct((C, 16), jnp.float32),
        scratch_types=[
            pltpu.VMEM((_EPAD,), jnp.int32),
            pltpu.VMEM((C, 16), jnp.float32),
            pltpu.SemaphoreType.DMA,
        ],
    )(_sc_hist_body)
    return kern(ei_flat_padded)


def _proj_body(x_ref, w_ref, o_ref):
    bf = jnp.bfloat16
    w = w_ref[...]
    o_ref[:, 0, :] = jnp.dot(x_ref[:, 0, :].astype(bf), w,
                             preferred_element_type=jnp.float32)
    o_ref[:, 1, :] = jnp.dot(x_ref[:, 1, :].astype(bf), w,
                             preferred_element_type=jnp.float32)


def _fused_body(P_ref, cnt_ref,
                W1o_ref, b1_ref, As1_ref, Ad1_ref, S1_ref,
                W2_ref, b2_ref, As2_ref, Ad2_ref, S2_ref,
                W3_ref, b3_ref, As3_ref, Ad3_ref, S3_ref, Mh_ref,
                R_ref, T_ref, G_ref,
                Wc1_ref, bc1_ref, Wc2_ref, bc2_ref,
                out_ref):
    f32 = jnp.float32
    dot = functools.partial(jnp.dot, preferred_element_type=f32)

    cnt_col = jnp.sum(cnt_ref[...], axis=1, keepdims=True)

    Tm = T_ref[...]
    Rm = R_ref[...]
    cnt_rows = dot(Tm, cnt_col)
    has_in = cnt_rows > 0.0

    def mix(h_first, h_second, As, Ad, S):
        ss0 = dot(h_first, As)
        ss1 = dot(h_second, As)
        sd1 = dot(h_second, Ad)
        e_self = _lrelu(ss1 + sd1)
        e_cross = _lrelu(ss0 + sd1)
        m = jnp.maximum(e_self, jnp.where(has_in, e_cross, -1e30))
        w_self = jnp.exp(e_self - m)
        w_cross = cnt_rows * jnp.exp(e_cross - m)
        denom = w_self + w_cross + 1e-16
        lam_s = dot(w_self / denom, S)
        lam_c = dot(w_cross / denom, S)
        return lam_s * h_second + lam_c * h_first

    oh = dot(Tm, W1o_ref[...])
    h0 = dot(Rm, P_ref[:, 0, :]) + oh
    h1 = dot(Rm, P_ref[:, 1, :]) + oh
    b1 = b1_ref[...]
    x0 = _elu(h0 + b1)
    x1 = _elu(mix(h0, h1, As1_ref[...], Ad1_ref[...], S1_ref[...]) + b1)

    bf = jnp.bfloat16
    W2 = W2_ref[...]
    H0 = dot(x0.astype(bf), W2)
    H1 = dot(x1.astype(bf), W2)
    b2 = b2_ref[...]
    x0 = _elu(H0 + b2)
    x1 = _elu(mix(H0, H1, As2_ref[...], Ad2_ref[...], S2_ref[...]) + b2)

    W3 = W3_ref[...]
    H0 = dot(x0.astype(bf), W3)
    H1 = dot(x1.astype(bf), W3)
    g1 = mix(H0, H1, As3_ref[...], Ad3_ref[...], S3_ref[...])
    b3 = b3_ref[...]
    x0 = _elu(dot(H0, Mh_ref[...]) + b3)
    x1 = _elu(dot(g1, Mh_ref[...]) + b3)

    Gm = G_ref[...]
    y0 = dot(Gm, x0)
    y1 = dot(Gm, x1)
    Wc1 = Wc1_ref[...]
    bc1 = bc1_ref[...]
    Wc2 = Wc2_ref[...]
    bc2 = bc2_ref[...]
    z0 = jnp.maximum(dot(y0, Wc1) + bc1, 0.0)
    z1 = jnp.maximum(dot(y1, Wc1) + bc1, 0.0)
    out_ref[:, 0, :] = jax.nn.sigmoid(dot(z0, Wc2) + bc2)
    out_ref[:, 1, :] = jax.nn.sigmoid(dot(z1, Wc2) + bc2)


def _head_mats(a_src, a_dst):
    heads, ch = a_src.shape
    eye = jnp.eye(heads, dtype=a_src.dtype)
    As = jnp.einsum('hc,hk->hck', a_src, eye).reshape(heads * ch, heads)
    Ad = jnp.einsum('hc,hk->hck', a_dst, eye).reshape(heads * ch, heads)
    S = jnp.repeat(eye, ch, axis=1)
    return As, Ad, S


def kernel(cnn_embeddings, edge_index, W1, a_src1, a_dst1, b1, W2, a_src2,
           a_dst2, b2, W3, a_src3, a_dst3, b3, Wc1, bc1, Wc2, bc2):
    f32 = jnp.float32
    emb = cnn_embeddings.astype(f32).reshape(2, HALF, EMBED).transpose(1, 0, 2)
    ei = edge_index.astype(jnp.int32)

    ei_flat = jnp.full((_EPAD,), C, jnp.int32).at[:2 * 210].set(
        ei.reshape(2 * 210))
    cnt_part = _sc_histogram(ei_flat)

    W1e = W1[:EMBED].astype(jnp.bfloat16)
    W1o = W1[EMBED:]
    As1, Ad1, S1 = _head_mats(a_src1, a_dst1)
    As2, Ad2, S2 = _head_mats(a_src2, a_dst2)
    As3, Ad3, S3 = _head_mats(a_src3, a_dst3)
    Mh = jnp.tile(jnp.eye(64, dtype=f32), (4, 1)) / 4.0

    R = jnp.repeat(jnp.eye(BP, dtype=f32), C, axis=0)
    T = jnp.tile(jnp.eye(C, dtype=f32), (BP, 1))
    G = R.T / C

    PROJ_BM = 128
    P = pl.pallas_call(
        _proj_body,
        grid=(HALF // PROJ_BM,),
        in_specs=[pl.BlockSpec((PROJ_BM, 2, EMBED), lambda i: (i, 0, 0)),
                  pl.BlockSpec((EMBED, F1), lambda i: (0, 0))],
        out_specs=pl.BlockSpec((PROJ_BM, 2, F1), lambda i: (i, 0, 0)),
        out_shape=jax.ShapeDtypeStruct((HALF, 2, F1), f32),
        compiler_params=pltpu.CompilerParams(
            dimension_semantics=("arbitrary",),
        ),
    )(emb, W1e)

    full = lambda shape: pl.BlockSpec(shape, lambda i: (0,) * len(shape))
    spec_P = pl.BlockSpec((BP, 2, F1), lambda i: (i, 0, 0))
    spec_out = pl.BlockSpec((BP, 2, C), lambda i: (i, 0, 0))

    operands = (
        P, cnt_part,
        W1o, b1.reshape(1, -1), As1, Ad1, S1,
        W2.astype(jnp.bfloat16), b2.reshape(1, -1), As2, Ad2, S2,
        W3.astype(jnp.bfloat16), b3.reshape(1, -1), As3, Ad3, S3, Mh,
        R, T, G,
        Wc1, bc1.reshape(1, -1), Wc2, bc2.reshape(1, -1),
    )
    in_specs = [spec_P] + [full(op.shape) for op in operands[1:]]

    out = pl.pallas_call(
        _fused_body,
        grid=(GRID,),
        in_specs=in_specs,
        out_specs=spec_out,
        out_shape=jax.ShapeDtypeStruct((HALF, 2, C), f32),
        compiler_params=pltpu.CompilerParams(
            dimension_semantics=("arbitrary",),
            vmem_limit_bytes=63 * 1024 * 1024,
        ),
    )(*operands)
    return out.transpose(1, 0, 2).reshape(B, C)


# speedup vs baseline: 1.0108x; 1.0108x over previous
import functools

import jax
import jax.numpy as jnp
from jax import lax
from jax.experimental import pallas as pl
from jax.experimental.pallas import tpu as pltpu
from jax.experimental.pallas import tpu_sc as plsc

C = 15
EMBED = 1024
B = 1024
HALF = B // 2
BP = 8
ROWS = BP * C
GRID = HALF // BP
F1 = 4096


def _lrelu(x):
    return jnp.where(x >= 0, x, 0.2 * x)


def _elu(x):
    return jnp.where(x > 0, x, jnp.exp(x) - 1.0)


_EPAD = 448


def _sc_hist_body(ei_hbm, out_hbm, idx_v, cnt_v, sem):
    wid = lax.axis_index("s") * 2 + lax.axis_index("c")

    @pl.when(wid == 0)
    def _():
        pltpu.async_copy(ei_hbm, idx_v, sem).wait()
        zeros = jnp.zeros((16,), jnp.float32)
        ones = jnp.full((16,), 1.0, jnp.float32)
        accs = [zeros] * C
        for j in range(_EPAD // 16):
            chunk = idx_v[pl.ds(j * 16, 16)]
            for v in range(C):
                accs[v] = accs[v] + jnp.where(chunk == v, ones, zeros)
        for v in range(C):
            cnt_v[v] = accs[v]
        pltpu.async_copy(cnt_v, out_hbm, sem).wait()


def _sc_histogram(ei_flat_padded):
    mesh = plsc.VectorSubcoreMesh(core_axis_name="c", subcore_axis_name="s")
    kern = functools.partial(
        pl.kernel,
        mesh=mesh,
        out_type=jax.ShapeDtypeStruct((C, 16), jnp.float32),
        scratch_types=[
            pltpu.VMEM((_EPAD,), jnp.int32),
            pltpu.VMEM((C, 16), jnp.float32),
            pltpu.SemaphoreType.DMA,
        ],
    )(_sc_hist_body)
    return kern(ei_flat_padded)


def _proj_body(x_ref, w_ref, o_ref):
    w = w_ref[...]
    o_ref[:, 0, :] = jnp.dot(x_ref[:, 0, :], w,
                             preferred_element_type=jnp.float32)
    o_ref[:, 1, :] = jnp.dot(x_ref[:, 1, :], w,
                             preferred_element_type=jnp.float32)


def _fused_body(P_ref, cnt_ref,
                W1o_ref, b1_ref, As1_ref, Ad1_ref, S1_ref,
                W2_ref, b2_ref, As2_ref, Ad2_ref, S2_ref,
                W3_ref, b3_ref, As3_ref, Ad3_ref, S3_ref, Mh_ref,
                R_ref, T_ref, G_ref,
                Wc1_ref, bc1_ref, Wc2_ref, bc2_ref,
                out_ref):
    f32 = jnp.float32
    dot = functools.partial(jnp.dot, preferred_element_type=f32)

    cnt_col = jnp.sum(cnt_ref[...], axis=1, keepdims=True)

    Tm = T_ref[...]
    Rm = R_ref[...]
    cnt_rows = dot(Tm, cnt_col)
    has_in = cnt_rows > 0.0

    def mix(h_first, h_second, As, Ad, S):
        ss0 = dot(h_first, As)
        ss1 = dot(h_second, As)
        sd1 = dot(h_second, Ad)
        e_self = _lrelu(ss1 + sd1)
        e_cross = _lrelu(ss0 + sd1)
        m = jnp.maximum(e_self, jnp.where(has_in, e_cross, -1e30))
        w_self = jnp.exp(e_self - m)
        w_cross = cnt_rows * jnp.exp(e_cross - m)
        denom = w_self + w_cross + 1e-16
        lam_s = dot(w_self / denom, S)
        lam_c = dot(w_cross / denom, S)
        return lam_s * h_second + lam_c * h_first

    oh = dot(Tm, W1o_ref[...])
    h0 = dot(Rm, P_ref[:, 0, :]) + oh
    h1 = dot(Rm, P_ref[:, 1, :]) + oh
    b1 = b1_ref[...]
    x0 = _elu(h0 + b1)
    x1 = _elu(mix(h0, h1, As1_ref[...], Ad1_ref[...], S1_ref[...]) + b1)

    H0 = dot(x0, W2_ref[...])
    H1 = dot(x1, W2_ref[...])
    b2 = b2_ref[...]
    x0 = _elu(H0 + b2)
    x1 = _elu(mix(H0, H1, As2_ref[...], Ad2_ref[...], S2_ref[...]) + b2)

    H0 = dot(x0, W3_ref[...])
    H1 = dot(x1, W3_ref[...])
    g1 = mix(H0, H1, As3_ref[...], Ad3_ref[...], S3_ref[...])
    b3 = b3_ref[...]
    x0 = _elu(dot(H0, Mh_ref[...]) + b3)
    x1 = _elu(dot(g1, Mh_ref[...]) + b3)

    Gm = G_ref[...]
    y0 = dot(Gm, x0)
    y1 = dot(Gm, x1)
    Wc1 = Wc1_ref[...]
    bc1 = bc1_ref[...]
    Wc2 = Wc2_ref[...]
    bc2 = bc2_ref[...]
    z0 = jnp.maximum(dot(y0, Wc1) + bc1, 0.0)
    z1 = jnp.maximum(dot(y1, Wc1) + bc1, 0.0)
    out_ref[:, 0, :] = jax.nn.sigmoid(dot(z0, Wc2) + bc2)
    out_ref[:, 1, :] = jax.nn.sigmoid(dot(z1, Wc2) + bc2)


def _head_mats(a_src, a_dst):
    heads, ch = a_src.shape
    eye = jnp.eye(heads, dtype=a_src.dtype)
    As = jnp.einsum('hc,hk->hck', a_src, eye).reshape(heads * ch, heads)
    Ad = jnp.einsum('hc,hk->hck', a_dst, eye).reshape(heads * ch, heads)
    S = jnp.repeat(eye, ch, axis=1)
    return As, Ad, S


def kernel(cnn_embeddings, edge_index, W1, a_src1, a_dst1, b1, W2, a_src2,
           a_dst2, b2, W3, a_src3, a_dst3, b3, Wc1, bc1, Wc2, bc2):
    f32 = jnp.float32
    emb = cnn_embeddings.astype(f32).reshape(2, HALF, EMBED).transpose(1, 0, 2)
    ei = edge_index.astype(jnp.int32)

    ei_flat = jnp.full((_EPAD,), C, jnp.int32).at[:2 * 210].set(
        ei.reshape(2 * 210))
    cnt_part = _sc_histogram(ei_flat)

    W1e = W1[:EMBED]
    W1o = W1[EMBED:]
    As1, Ad1, S1 = _head_mats(a_src1, a_dst1)
    As2, Ad2, S2 = _head_mats(a_src2, a_dst2)
    As3, Ad3, S3 = _head_mats(a_src3, a_dst3)
    Mh = jnp.tile(jnp.eye(64, dtype=f32), (4, 1)) / 4.0

    R = jnp.repeat(jnp.eye(BP, dtype=f32), C, axis=0)
    T = jnp.tile(jnp.eye(C, dtype=f32), (BP, 1))
    G = R.T / C

    PROJ_BM = 128
    P = pl.pallas_call(
        _proj_body,
        grid=(HALF // PROJ_BM,),
        in_specs=[pl.BlockSpec((PROJ_BM, 2, EMBED), lambda i: (i, 0, 0)),
                  pl.BlockSpec((EMBED, F1), lambda i: (0, 0))],
        out_specs=pl.BlockSpec((PROJ_BM, 2, F1), lambda i: (i, 0, 0)),
        out_shape=jax.ShapeDtypeStruct((HALF, 2, F1), f32),
        compiler_params=pltpu.CompilerParams(
            dimension_semantics=("arbitrary",),
        ),
    )(emb, W1e)

    full = lambda shape: pl.BlockSpec(shape, lambda i: (0,) * len(shape))
    spec_P = pl.BlockSpec((BP, 2, F1), lambda i: (i, 0, 0))
    spec_out = pl.BlockSpec((BP, 2, C), lambda i: (i, 0, 0))

    operands = (
        P, cnt_part,
        W1o, b1.reshape(1, -1), As1, Ad1, S1,
        W2, b2.reshape(1, -1), As2, Ad2, S2,
        W3, b3.reshape(1, -1), As3, Ad3, S3, Mh,
        R, T, G,
        Wc1, bc1.reshape(1, -1), Wc2, bc2.reshape(1, -1),
    )
    in_specs = [spec_P] + [full(op.shape) for op in operands[1:]]

    out = pl.pallas_call(
        _fused_body,
        grid=(GRID,),
        in_specs=in_specs,
        out_specs=spec_out,
        out_shape=jax.ShapeDtypeStruct((HALF, 2, C), f32),
        compiler_params=pltpu.CompilerParams(
            dimension_semantics=("arbitrary",),
            vmem_limit_bytes=63 * 1024 * 1024,
        ),
    )(*operands)
    return out.transpose(1, 0, 2).reshape(B, C)


# speedup vs baseline: 1.1109x; 1.0991x over previous
import functools

import jax
import jax.numpy as jnp
from jax import lax
from jax.experimental import pallas as pl
from jax.experimental.pallas import tpu as pltpu
from jax.experimental.pallas import tpu_sc as plsc

C = 15
EMBED = 1024
B = 1024
HALF = B // 2
BP = 8
ROWS = BP * C
GRID = HALF // BP
F1 = 4096


def _lrelu(x):
    return jnp.where(x >= 0, x, 0.2 * x)


def _elu(x):
    return jnp.where(x > 0, x, jnp.exp(x) - 1.0)


_EPAD = 448


def _sc_hist_body(ei_hbm, out_hbm, idx_v, cnt_v, sem):
    wid = lax.axis_index("s") * 2 + lax.axis_index("c")

    @pl.when(wid == 0)
    def _():
        pltpu.async_copy(ei_hbm, idx_v, sem).wait()
        zeros = jnp.zeros((16,), jnp.float32)
        ones = jnp.full((16,), 1.0, jnp.float32)
        accs = [zeros] * C
        for j in range(_EPAD // 16):
            chunk = idx_v[pl.ds(j * 16, 16)]
            for v in range(C):
                accs[v] = accs[v] + jnp.where(chunk == v, ones, zeros)
        for v in range(C):
            cnt_v[v] = accs[v]
        pltpu.async_copy(cnt_v, out_hbm, sem).wait()


def _sc_histogram(ei_flat_padded):
    mesh = plsc.VectorSubcoreMesh(core_axis_name="c", subcore_axis_name="s")
    kern = functools.partial(
        pl.kernel,
        mesh=mesh,
        out_type=jax.ShapeDtypeStruct((C, 16), jnp.float32),
        scratch_types=[
            pltpu.VMEM((_EPAD,), jnp.int32),
            pltpu.VMEM((C, 16), jnp.float32),
            pltpu.SemaphoreType.DMA,
        ],
    )(_sc_hist_body)
    return kern(ei_flat_padded)


def _proj_body(x_ref, w_ref, o_ref):
    w = w_ref[...]
    o_ref[:, 0, :] = jnp.dot(x_ref[:, 0, :], w,
                             preferred_element_type=jnp.float32)
    o_ref[:, 1, :] = jnp.dot(x_ref[:, 1, :], w,
                             preferred_element_type=jnp.float32)


def _fused_body(P_ref, cnt_ref,
                W1o_ref, b1_ref, AsAd1_ref, S1_ref,
                W2_ref, b2_ref, AsAd2_ref, S2_ref,
                W3_ref, b3_ref, AsAd3_ref, S3_ref, Mh_ref,
                R_ref, T_ref, G_ref,
                Wc1_ref, bc1_ref, Wc2_ref, bc2_ref,
                out_ref):
    f32 = jnp.float32
    dot = functools.partial(jnp.dot, preferred_element_type=f32)

    cnt_col = jnp.sum(cnt_ref[...], axis=1, keepdims=True)

    Tm = T_ref[...]
    Rm = R_ref[...]
    cnt_rows = dot(Tm, cnt_col)
    has_in = cnt_rows > 0.0

    def mix(h_first, h_second, AsAd, S):
        heads = AsAd.shape[1] // 2
        sa0 = dot(h_first, AsAd)
        sa1 = dot(h_second, AsAd)
        ss0 = sa0[:, :heads]
        ss1 = sa1[:, :heads]
        sd1 = sa1[:, heads:]
        e_self = _lrelu(ss1 + sd1)
        e_cross = _lrelu(ss0 + sd1)
        m = jnp.maximum(e_self, jnp.where(has_in, e_cross, -1e30))
        w_self = jnp.exp(e_self - m)
        w_cross = cnt_rows * jnp.exp(e_cross - m)
        denom = w_self + w_cross + 1e-16
        lam_s = dot(w_self / denom, S)
        return h_first + lam_s * (h_second - h_first)

    oh = dot(Tm, W1o_ref[...])
    h0 = dot(Rm, P_ref[:, 0, :]) + oh
    h1 = dot(Rm, P_ref[:, 1, :]) + oh
    b1 = b1_ref[...]
    x0 = _elu(h0 + b1)
    x1 = _elu(mix(h0, h1, AsAd1_ref[...], S1_ref[...]) + b1)

    H0 = dot(x0, W2_ref[...])
    H1 = dot(x1, W2_ref[...])
    b2 = b2_ref[...]
    x0 = _elu(H0 + b2)
    x1 = _elu(mix(H0, H1, AsAd2_ref[...], S2_ref[...]) + b2)

    H0 = dot(x0, W3_ref[...])
    H1 = dot(x1, W3_ref[...])
    g1 = mix(H0, H1, AsAd3_ref[...], S3_ref[...])
    b3 = b3_ref[...]
    x0 = _elu(dot(H0, Mh_ref[...]) + b3)
    x1 = _elu(dot(g1, Mh_ref[...]) + b3)

    Gm = G_ref[...]
    y0 = dot(Gm, x0)
    y1 = dot(Gm, x1)
    Wc1 = Wc1_ref[...]
    bc1 = bc1_ref[...]
    Wc2 = Wc2_ref[...]
    bc2 = bc2_ref[...]
    z0 = jnp.maximum(dot(y0, Wc1) + bc1, 0.0)
    z1 = jnp.maximum(dot(y1, Wc1) + bc1, 0.0)
    out_ref[:, 0, :] = jax.nn.sigmoid(dot(z0, Wc2) + bc2)
    out_ref[:, 1, :] = jax.nn.sigmoid(dot(z1, Wc2) + bc2)


def _head_mats(a_src, a_dst):
    heads, ch = a_src.shape
    eye = jnp.eye(heads, dtype=a_src.dtype)
    As = jnp.einsum('hc,hk->hck', a_src, eye).reshape(heads * ch, heads)
    Ad = jnp.einsum('hc,hk->hck', a_dst, eye).reshape(heads * ch, heads)
    S = jnp.repeat(eye, ch, axis=1)
    return jnp.concatenate([As, Ad], axis=1), S


def kernel(cnn_embeddings, edge_index, W1, a_src1, a_dst1, b1, W2, a_src2,
           a_dst2, b2, W3, a_src3, a_dst3, b3, Wc1, bc1, Wc2, bc2):
    f32 = jnp.float32
    emb = cnn_embeddings.astype(f32).reshape(2, HALF, EMBED).transpose(1, 0, 2)
    ei = edge_index.astype(jnp.int32)

    ei_flat = jnp.full((_EPAD,), C, jnp.int32).at[:2 * 210].set(
        ei.reshape(2 * 210))
    cnt_part = _sc_histogram(ei_flat)

    W1e = W1[:EMBED]
    W1o = W1[EMBED:]
    AsAd1, S1 = _head_mats(a_src1, a_dst1)
    AsAd2, S2 = _head_mats(a_src2, a_dst2)
    AsAd3, S3 = _head_mats(a_src3, a_dst3)
    Mh = jnp.tile(jnp.eye(64, dtype=f32), (4, 1)) / 4.0

    R = jnp.repeat(jnp.eye(BP, dtype=f32), C, axis=0)
    T = jnp.tile(jnp.eye(C, dtype=f32), (BP, 1))
    G = R.T / C

    PROJ_BM = 128
    P = pl.pallas_call(
        _proj_body,
        grid=(HALF // PROJ_BM,),
        in_specs=[pl.BlockSpec((PROJ_BM, 2, EMBED), lambda i: (i, 0, 0)),
                  pl.BlockSpec((EMBED, F1), lambda i: (0, 0))],
        out_specs=pl.BlockSpec((PROJ_BM, 2, F1), lambda i: (i, 0, 0)),
        out_shape=jax.ShapeDtypeStruct((HALF, 2, F1), f32),
        compiler_params=pltpu.CompilerParams(
            dimension_semantics=("arbitrary",),
        ),
    )(emb, W1e)

    full = lambda shape: pl.BlockSpec(shape, lambda i: (0,) * len(shape))
    spec_P = pl.BlockSpec((BP, 2, F1), lambda i: (i, 0, 0))
    spec_out = pl.BlockSpec((BP, 2, C), lambda i: (i, 0, 0))

    operands = (
        P, cnt_part,
        W1o, b1.reshape(1, -1), AsAd1, S1,
        W2, b2.reshape(1, -1), AsAd2, S2,
        W3, b3.reshape(1, -1), AsAd3, S3, Mh,
        R, T, G,
        Wc1, bc1.reshape(1, -1), Wc2, bc2.reshape(1, -1),
    )
    in_specs = [spec_P] + [full(op.shape) for op in operands[1:]]

    out = pl.pallas_call(
        _fused_body,
        grid=(GRID,),
        in_specs=in_specs,
        out_specs=spec_out,
        out_shape=jax.ShapeDtypeStruct((HALF, 2, C), f32),
        compiler_params=pltpu.CompilerParams(
            dimension_semantics=("arbitrary",),
            vmem_limit_bytes=63 * 1024 * 1024,
        ),
    )(*operands)
    return out.transpose(1, 0, 2).reshape(B, C)
